# baseline (speedup 1.0000x reference)
import functools

import jax
import jax.numpy as jnp
from jax import lax
from jax.experimental import pallas as pl
from jax.experimental.pallas import tpu as pltpu
from jax.experimental.pallas import tpu_sc as plsc

_NEGS = 5
_SCALING = 0.1
_WIN = 128


def _tc_transpose(tt):
    D, V = tt.shape
    BK = 8192
    grid = (V + BK - 1) // BK

    def body(t_r, o_r):
        o_r[...] = jnp.transpose(t_r[...], (1, 0))

    return pl.pallas_call(
        body,
        grid=(grid,),
        in_specs=[pl.BlockSpec((D, BK), lambda i: (0, i))],
        out_specs=pl.BlockSpec((BK, D), lambda i: (i, 0)),
        out_shape=jax.ShapeDtypeStruct((V, D), jnp.float32),
    )(tt)


def _sc_gather_pair(idx, tab1, tab2):
    n = idx.shape[1]
    D = tab1.shape[1]
    f32 = jnp.float32
    mesh = plsc.VectorSubcoreMesh(core_axis_name="c", subcore_axis_name="s")
    out_type = [jax.ShapeDtypeStruct((n, D), f32),
                jax.ShapeDtypeStruct((n, D), f32)]
    ispec = pl.BlockSpec((1, _WIN), lambda i: (0, i))
    ospec = pl.BlockSpec((_WIN, D), lambda i: (i, 0))

    @functools.partial(
        pl.kernel, out_type=out_type, mesh=mesh,
        compiler_params=pltpu.CompilerParams(use_tc_tiling_on_sc=False))
    def gk(idx_h, t1_h, t2_h, o1_h, o2_h):
        def body(i_v, o1_v, o2_v):
            pltpu.sync_copy(t1_h.at[i_v.at[0]], o1_v)
            pltpu.sync_copy(t2_h.at[i_v.at[0]], o2_v)

        pltpu.emit_pipeline(
            body, grid=(n // _WIN,),
            in_specs=[ispec], out_specs=[ospec, ospec],
            core_axis_name=("c", "s"), dimension_semantics=(pltpu.PARALLEL,),
        )(idx_h, o1_h, o2_h)

    return gk(idx, tab1, tab2)


def _sc_gather_one(idx, tab):
    n = idx.shape[1]
    D = tab.shape[1]
    mesh = plsc.VectorSubcoreMesh(core_axis_name="c", subcore_axis_name="s")
    out_type = jax.ShapeDtypeStruct((n, D), jnp.float32)
    ispec = pl.BlockSpec((1, _WIN), lambda i: (0, i))
    ospec = pl.BlockSpec((_WIN, D), lambda i: (i, 0))

    @functools.partial(
        pl.kernel, out_type=out_type, mesh=mesh,
        compiler_params=pltpu.CompilerParams(use_tc_tiling_on_sc=False))
    def gk(idx_h, t_h, o_h):
        def body(i_v, o_v):
            pltpu.sync_copy(t_h.at[i_v.at[0]], o_v)

        pltpu.emit_pipeline(
            body, grid=(n // _WIN,),
            in_specs=[ispec], out_specs=[ospec],
            core_axis_name=("c", "s"), dimension_semantics=(pltpu.PARALLEL,),
        )(idx_h, o_h)

    return gk(idx, tab)


def _sc_gather_big(out_idx, noise_idx, tab):
    n_out = out_idx.shape[1]
    n_noise = noise_idx.shape[1]
    D = tab.shape[1]
    f32 = jnp.float32
    mesh = plsc.VectorSubcoreMesh(core_axis_name="c", subcore_axis_name="s")
    out_type = [jax.ShapeDtypeStruct((n_out, D), f32),
                jax.ShapeDtypeStruct((n_noise, D), f32)]
    ispec = pl.BlockSpec((1, _WIN), lambda i: (0, i))
    ospec = pl.BlockSpec((_WIN, D), lambda i: (i, 0))

    @functools.partial(
        pl.kernel, out_type=out_type, mesh=mesh,
        compiler_params=pltpu.CompilerParams(use_tc_tiling_on_sc=False))
    def gk(out_idx_h, noise_idx_h, t_h, mu_h, nz_h):
        def body(i_v, o_v):
            pltpu.sync_copy(t_h.at[i_v.at[0]], o_v)

        pltpu.emit_pipeline(
            body, grid=(n_out // _WIN,),
            in_specs=[ispec], out_specs=[ospec],
            core_axis_name=("c", "s"), dimension_semantics=(pltpu.PARALLEL,),
        )(out_idx_h, mu_h)
        pltpu.emit_pipeline(
            body, grid=(n_noise // _WIN,),
            in_specs=[ispec], out_specs=[ospec],
            core_axis_name=("c", "s"), dimension_semantics=(pltpu.PARALLEL,),
        )(noise_idx_h, nz_h)

    return gk(out_idx, noise_idx, tab)


def _tc_math(mu_in, rho_in, eps_in, covf, covw, wT, bvec,
             mu_out, rho_out, eps_out, noise_v, B, W, D):
    GB = 128
    nblocks = B // GB
    GW = GB * W
    f32 = jnp.float32
    hi = lax.Precision.HIGHEST

    def body(mu_in_r, rho_in_r, eps_in_r, cov_r, covw_r, wT_r, b_r,
             mu_out_r, rho_out_r, eps_out_r, noise_r, kl_r, lik_r):
        @pl.when(pl.program_id(0) == 0)
        def _():
            kl_r[...] = jnp.zeros((1, 1), f32)
            lik_r[...] = jnp.zeros((1, 1), f32)

        mu_in = mu_in_r[...]
        rho_in = rho_in_r[...]
        eps_in = eps_in_r[...]
        cov = cov_r[...]
        covw = covw_r[...]
        wT = wT_r[...]
        bb = b_r[...]

        y = covw[0:1, :] + cov * (covw[1:2, :] - covw[0:1, :])
        sig_in = jnp.log(jnp.exp(rho_in) + 1.0)
        h = (jnp.dot(mu_in, wT[0:D, :], precision=hi, preferred_element_type=f32)
             + jnp.dot(y, wT[D:2 * D, :], precision=hi, preferred_element_type=f32)
             + bb)
        w_in = jnp.tanh(h) + _SCALING * sig_in * eps_in
        post_in = -0.5 * jnp.sum(eps_in * eps_in) - jnp.sum(jnp.log(sig_in))
        wsq = w_in * w_in
        prior_in = jnp.sum(jnp.log(0.5 * jnp.exp(-wsq / 2.0)
                                   + 0.5 * jnp.exp(-wsq / 0.08)))
        kl = W * (post_in - prior_in)

        rowi = lax.broadcasted_iota(jnp.int32, (GW, GB), 0) // W
        colj = lax.broadcasted_iota(jnp.int32, (GW, GB), 1)
        sel = (rowi == colj).astype(f32)
        w_inb = jnp.dot(sel, w_in, precision=hi, preferred_element_type=f32)

        mu_out = mu_out_r[...]
        rho_out = rho_out_r[...]
        eps_out = eps_out_r[...]
        sig_out = jnp.log(jnp.exp(rho_out) + 1.0)
        w_out = mu_out + _SCALING * sig_out * eps_out
        post_out = (-0.5 * jnp.sum(eps_out * eps_out)
                    - jnp.sum(jnp.log(sig_out)))
        wsq_o = w_out * w_out
        prior_out = jnp.sum(jnp.log(0.5 * jnp.exp(-wsq_o / 2.0)
                                    + 0.5 * jnp.exp(-wsq_o / 0.08)))
        kl += post_out - prior_out

        s = jnp.sum(w_inb * w_out, axis=1, keepdims=True)
        lik = jnp.sum(jnp.log(jax.nn.sigmoid(s)))
        ls = jnp.float32(0.0)
        for j in range(_NEGS):
            nj = noise_r[:, j * D:(j + 1) * D]
            sj = jnp.sum(w_inb * nj, axis=1, keepdims=True)
            ls += jnp.sum(jnp.log(jax.nn.sigmoid(-sj)))
        lik += ls / _NEGS

        kl_r[...] += kl.reshape(1, 1)
        lik_r[...] += lik.reshape(1, 1)

    acc_spec = pl.BlockSpec((1, 1), lambda i: (0, 0))
    kl_sum, lik_sum = pl.pallas_call(
        body,
        grid=(nblocks,),
        in_specs=[
            pl.BlockSpec((GB, D), lambda i: (i, 0)),
            pl.BlockSpec((GB, D), lambda i: (i, 0)),
            pl.BlockSpec((GB, D), lambda i: (i, 0)),
            pl.BlockSpec((GB, 1), lambda i: (i, 0)),
            pl.BlockSpec((2, D), lambda i: (0, 0)),
            pl.BlockSpec((2 * D, D), lambda i: (0, 0)),
            pl.BlockSpec((1, D), lambda i: (0, 0)),
            pl.BlockSpec((GW, D), lambda i: (i, 0)),
            pl.BlockSpec((GW, D), lambda i: (i, 0)),
            pl.BlockSpec((GW, D), lambda i: (i, 0)),
            pl.BlockSpec((GW, _NEGS * D), lambda i: (i, 0)),
        ],
        out_specs=[acc_spec, acc_spec],
        out_shape=[jax.ShapeDtypeStruct((1, 1), f32)] * 2,
    )(mu_in, rho_in, eps_in, covf, covw, wT, bvec,
      mu_out, rho_out, eps_out, noise_v)
    return kl_sum, lik_sum


def kernel(inputs, outputs, covars, wt, batch_num, in_embed_w, out_embed_w,
           in_rho_w, out_rho_w, covariates_w, linear_w, linear_b):
    B, W = outputs.shape
    V, D = in_embed_w.shape

    key = jax.random.key(42)
    k1, k2, k3 = jax.random.split(key, 3)
    eps_in = jax.random.normal(k1, (B, 1, D), jnp.float32).reshape(B, D)
    eps_out = jax.random.normal(k2, (B, W, D), jnp.float32).reshape(B * W, D)
    noise_idx = jax.random.randint(k3, (B * W, _NEGS), 0, V)

    out_idx = outputs.astype(jnp.int32).reshape(1, B * W)
    nz_idx = noise_idx.astype(jnp.int32).reshape(1, B * W * _NEGS)
    in_idx = inputs.astype(jnp.int32).reshape(1, B)

    oe_rm = _tc_transpose(out_embed_w.T)
    mu_out_d, noise_d = _sc_gather_big(out_idx, nz_idx, oe_rm)
    or_rm = _tc_transpose(out_rho_w.T)
    ie_rm = _tc_transpose(in_embed_w.T)
    ir_rm = _tc_transpose(in_rho_w.T)
    rho_out_d = _sc_gather_one(out_idx, or_rm)
    mu_in_d, rho_in_d = _sc_gather_pair(in_idx, ie_rm, ir_rm)

    noise_v = noise_d.reshape(B * W, _NEGS * D)
    covf = covars.astype(jnp.float32).reshape(B, 1)
    wT = linear_w.T
    bvec = linear_b.reshape(1, D)

    kl_sum, lik_sum = _tc_math(mu_in_d, rho_in_d, eps_in, covf, covariates_w,
                               wT, bvec, mu_out_d, rho_out_d, eps_out,
                               noise_v, B, W, D)
    loss = (wt[0] * kl_sum[0, 0] - lik_sum[0, 0]) / (B * W)
    return loss


# speedup vs baseline: 1.8861x; 1.8861x over previous
import functools

import jax
import jax.numpy as jnp
from jax import lax
from jax.experimental import pallas as pl
from jax.experimental.pallas import tpu as pltpu
from jax.experimental.pallas import tpu_sc as plsc

_NEGS = 5
_SCALING = 0.1
_WIN = 128


_TBK = 8192
_TS = _TBK // 2


def _tc_transpose(tt):
    D, V = tt.shape
    grid = (V + _TBK - 1) // _TBK

    def body(t_r, o_r):
        o_r[:, 0:D] = jnp.transpose(t_r[:, 0:_TS], (1, 0))
        o_r[:, D:2 * D] = jnp.transpose(t_r[:, _TS:_TBK], (1, 0))

    out = pl.pallas_call(
        body,
        grid=(grid,),
        in_specs=[pl.BlockSpec((D, _TBK), lambda i: (0, i))],
        out_specs=pl.BlockSpec((_TS, 2 * D), lambda i: (i, 0)),
        out_shape=jax.ShapeDtypeStruct((grid * _TS, 2 * D), jnp.float32),
    )(tt)
    return out.reshape(2 * grid * _TS, D)


def _pair_idx(idx):
    q = idx // _TS
    r = idx - q * _TS
    return (q // 2) * (2 * _TS) + 2 * r + (q & 1)


def _sc_gather_pair(idx, tab1, tab2):
    n = idx.shape[1]
    D = tab1.shape[1]
    f32 = jnp.float32
    mesh = plsc.VectorSubcoreMesh(core_axis_name="c", subcore_axis_name="s")
    out_type = [jax.ShapeDtypeStruct((n, D), f32),
                jax.ShapeDtypeStruct((n, D), f32)]
    ispec = pl.BlockSpec((1, _WIN), lambda i: (0, i))
    ospec = pl.BlockSpec((_WIN, D), lambda i: (i, 0))

    @functools.partial(
        pl.kernel, out_type=out_type, mesh=mesh,
        compiler_params=pltpu.CompilerParams(use_tc_tiling_on_sc=False))
    def gk(idx_h, t1_h, t2_h, o1_h, o2_h):
        def body(i_v, o1_v, o2_v):
            pltpu.sync_copy(t1_h.at[i_v.at[0]], o1_v)
            pltpu.sync_copy(t2_h.at[i_v.at[0]], o2_v)

        pltpu.emit_pipeline(
            body, grid=(n // _WIN,),
            in_specs=[ispec], out_specs=[ospec, ospec],
            core_axis_name=("c", "s"), dimension_semantics=(pltpu.PARALLEL,),
        )(idx_h, o1_h, o2_h)

    return gk(idx, tab1, tab2)


def _sc_gather_one(idx, tab):
    n = idx.shape[1]
    D = tab.shape[1]
    mesh = plsc.VectorSubcoreMesh(core_axis_name="c", subcore_axis_name="s")
    out_type = jax.ShapeDtypeStruct((n, D), jnp.float32)
    ispec = pl.BlockSpec((1, _WIN), lambda i: (0, i))
    ospec = pl.BlockSpec((_WIN, D), lambda i: (i, 0))

    @functools.partial(
        pl.kernel, out_type=out_type, mesh=mesh,
        compiler_params=pltpu.CompilerParams(use_tc_tiling_on_sc=False))
    def gk(idx_h, t_h, o_h):
        def body(i_v, o_v):
            pltpu.sync_copy(t_h.at[i_v.at[0]], o_v)

        pltpu.emit_pipeline(
            body, grid=(n // _WIN,),
            in_specs=[ispec], out_specs=[ospec],
            core_axis_name=("c", "s"), dimension_semantics=(pltpu.PARALLEL,),
        )(idx_h, o_h)

    return gk(idx, tab)


def _sc_gather_big(out_idx, noise_idx, tab):
    n_out = out_idx.shape[1]
    n_noise = noise_idx.shape[1]
    D = tab.shape[1]
    f32 = jnp.float32
    mesh = plsc.VectorSubcoreMesh(core_axis_name="c", subcore_axis_name="s")
    out_type = [jax.ShapeDtypeStruct((n_out, D), f32),
                jax.ShapeDtypeStruct((n_noise, D), f32)]
    ispec = pl.BlockSpec((1, _WIN), lambda i: (0, i))
    ospec = pl.BlockSpec((_WIN, D), lambda i: (i, 0))

    @functools.partial(
        pl.kernel, out_type=out_type, mesh=mesh,
        compiler_params=pltpu.CompilerParams(use_tc_tiling_on_sc=False))
    def gk(out_idx_h, noise_idx_h, t_h, mu_h, nz_h):
        def body(i_v, o_v):
            pltpu.sync_copy(t_h.at[i_v.at[0]], o_v)

        pltpu.emit_pipeline(
            body, grid=(n_out // _WIN,),
            in_specs=[ispec], out_specs=[ospec],
            core_axis_name=("c", "s"), dimension_semantics=(pltpu.PARALLEL,),
        )(out_idx_h, mu_h)
        pltpu.emit_pipeline(
            body, grid=(n_noise // _WIN,),
            in_specs=[ispec], out_specs=[ospec],
            core_axis_name=("c", "s"), dimension_semantics=(pltpu.PARALLEL,),
        )(noise_idx_h, nz_h)

    return gk(out_idx, noise_idx, tab)


def _tc_math(mu_in, rho_in, eps_in, covf, covw, wT, bvec,
             mu_out, rho_out, eps_out, noise_v, B, W, D):
    GB = 128
    nblocks = B // GB
    GW = GB * W
    f32 = jnp.float32
    hi = lax.Precision.HIGHEST

    def body(mu_in_r, rho_in_r, eps_in_r, cov_r, covw_r, wT_r, b_r,
             mu_out_r, rho_out_r, eps_out_r, noise_r, kl_r, lik_r):
        @pl.when(pl.program_id(0) == 0)
        def _():
            kl_r[...] = jnp.zeros((1, 1), f32)
            lik_r[...] = jnp.zeros((1, 1), f32)

        mu_in = mu_in_r[...]
        rho_in = rho_in_r[...]
        eps_in = eps_in_r[...]
        cov = cov_r[...]
        covw = covw_r[...]
        wT = wT_r[...]
        bb = b_r[...]

        y = covw[0:1, :] + cov * (covw[1:2, :] - covw[0:1, :])
        sig_in = jnp.log(jnp.exp(rho_in) + 1.0)
        h = (jnp.dot(mu_in, wT[0:D, :], precision=hi, preferred_element_type=f32)
             + jnp.dot(y, wT[D:2 * D, :], precision=hi, preferred_element_type=f32)
             + bb)
        w_in = jnp.tanh(h) + _SCALING * sig_in * eps_in
        post_in = -0.5 * jnp.sum(eps_in * eps_in) - jnp.sum(jnp.log(sig_in))
        wsq = w_in * w_in
        prior_in = jnp.sum(jnp.log(0.5 * jnp.exp(-wsq / 2.0)
                                   + 0.5 * jnp.exp(-wsq / 0.08)))
        kl = W * (post_in - prior_in)

        rowi = lax.broadcasted_iota(jnp.int32, (GW, GB), 0) // W
        colj = lax.broadcasted_iota(jnp.int32, (GW, GB), 1)
        sel = (rowi == colj).astype(f32)
        w_inb = jnp.dot(sel, w_in, precision=hi, preferred_element_type=f32)

        mu_out = mu_out_r[...]
        rho_out = rho_out_r[...]
        eps_out = eps_out_r[...]
        sig_out = jnp.log(jnp.exp(rho_out) + 1.0)
        w_out = mu_out + _SCALING * sig_out * eps_out
        post_out = (-0.5 * jnp.sum(eps_out * eps_out)
                    - jnp.sum(jnp.log(sig_out)))
        wsq_o = w_out * w_out
        prior_out = jnp.sum(jnp.log(0.5 * jnp.exp(-wsq_o / 2.0)
                                    + 0.5 * jnp.exp(-wsq_o / 0.08)))
        kl += post_out - prior_out

        s = jnp.sum(w_inb * w_out, axis=1, keepdims=True)
        lik = jnp.sum(jnp.log(jax.nn.sigmoid(s)))
        ls = jnp.float32(0.0)
        for j in range(_NEGS):
            nj = noise_r[:, j * D:(j + 1) * D]
            sj = jnp.sum(w_inb * nj, axis=1, keepdims=True)
            ls += jnp.sum(jnp.log(jax.nn.sigmoid(-sj)))
        lik += ls / _NEGS

        kl_r[...] += kl.reshape(1, 1)
        lik_r[...] += lik.reshape(1, 1)

    acc_spec = pl.BlockSpec((1, 1), lambda i: (0, 0))
    kl_sum, lik_sum = pl.pallas_call(
        body,
        grid=(nblocks,),
        in_specs=[
            pl.BlockSpec((GB, D), lambda i: (i, 0)),
            pl.BlockSpec((GB, D), lambda i: (i, 0)),
            pl.BlockSpec((GB, D), lambda i: (i, 0)),
            pl.BlockSpec((GB, 1), lambda i: (i, 0)),
            pl.BlockSpec((2, D), lambda i: (0, 0)),
            pl.BlockSpec((2 * D, D), lambda i: (0, 0)),
            pl.BlockSpec((1, D), lambda i: (0, 0)),
            pl.BlockSpec((GW, D), lambda i: (i, 0)),
            pl.BlockSpec((GW, D), lambda i: (i, 0)),
            pl.BlockSpec((GW, D), lambda i: (i, 0)),
            pl.BlockSpec((GW, _NEGS * D), lambda i: (i, 0)),
        ],
        out_specs=[acc_spec, acc_spec],
        out_shape=[jax.ShapeDtypeStruct((1, 1), f32)] * 2,
    )(mu_in, rho_in, eps_in, covf, covw, wT, bvec,
      mu_out, rho_out, eps_out, noise_v)
    return kl_sum, lik_sum


def kernel(inputs, outputs, covars, wt, batch_num, in_embed_w, out_embed_w,
           in_rho_w, out_rho_w, covariates_w, linear_w, linear_b):
    B, W = outputs.shape
    V, D = in_embed_w.shape

    key = jax.random.key(42)
    k1, k2, k3 = jax.random.split(key, 3)
    eps_in = jax.random.normal(k1, (B, D), jnp.float32)
    eps_out = jax.random.normal(k2, (B * W, D), jnp.float32)
    noise_idx = jax.random.randint(k3, (B * W, _NEGS), 0, V)

    out_idx = _pair_idx(outputs.astype(jnp.int32)).reshape(1, B * W)
    nz_idx = _pair_idx(noise_idx.astype(jnp.int32)).reshape(1, B * W * _NEGS)
    in_idx = _pair_idx(inputs.astype(jnp.int32)).reshape(1, B)

    oe_rm = _tc_transpose(out_embed_w.T)
    mu_out_d, noise_d = _sc_gather_big(out_idx, nz_idx, oe_rm)
    or_rm = _tc_transpose(out_rho_w.T)
    ie_rm = _tc_transpose(in_embed_w.T)
    ir_rm = _tc_transpose(in_rho_w.T)
    rho_out_d = _sc_gather_one(out_idx, or_rm)
    mu_in_d, rho_in_d = _sc_gather_pair(in_idx, ie_rm, ir_rm)

    noise_v = noise_d.reshape(B * W, _NEGS * D)
    covf = covars.astype(jnp.float32).reshape(B, 1)
    wT = linear_w.T
    bvec = linear_b.reshape(1, D)

    kl_sum, lik_sum = _tc_math(mu_in_d, rho_in_d, eps_in, covf, covariates_w,
                               wT, bvec, mu_out_d, rho_out_d, eps_out,
                               noise_v, B, W, D)
    loss = (wt[0] * kl_sum[0, 0] - lik_sum[0, 0]) / (B * W)
    return loss


# speedup vs baseline: 1.9576x; 1.0379x over previous
import functools

import jax
import jax.numpy as jnp
from jax import lax
from jax.experimental import pallas as pl
from jax.experimental.pallas import tpu as pltpu
from jax.experimental.pallas import tpu_sc as plsc

_NEGS = 5
_SCALING = 0.1
_WIN = 128
_TBK = 4096

_PAR = pltpu.CompilerParams(dimension_semantics=("parallel",))


def _tc_transpose_fused(at, bt):
    D, V = at.shape
    grid = (V + _TBK - 1) // _TBK

    def body(a_r, b_r, o_r):
        o_r[:, 0:D] = jnp.transpose(a_r[...], (1, 0))
        o_r[:, D:2 * D] = jnp.transpose(b_r[...], (1, 0))

    return pl.pallas_call(
        body,
        grid=(grid,),
        in_specs=[pl.BlockSpec((D, _TBK), lambda i: (0, i)),
                  pl.BlockSpec((D, _TBK), lambda i: (0, i))],
        out_specs=pl.BlockSpec((_TBK, 2 * D), lambda i: (i, 0)),
        out_shape=jax.ShapeDtypeStruct((grid * _TBK, 2 * D), jnp.float32),
        compiler_params=_PAR,
    )(at, bt)


def _sc_gather_out(mu_idx, rho_idx, nz_idx, tab):
    n = mu_idx.shape[1]
    n_nz = nz_idx.shape[1]
    D = tab.shape[1]
    f32 = jnp.float32
    mesh = plsc.VectorSubcoreMesh(core_axis_name="c", subcore_axis_name="s")
    out_type = [jax.ShapeDtypeStruct((n, D), f32),
                jax.ShapeDtypeStruct((n, D), f32),
                jax.ShapeDtypeStruct((n_nz, D), f32)]
    ispec = pl.BlockSpec((1, _WIN), lambda i: (0, i))
    ospec = pl.BlockSpec((_WIN, D), lambda i: (i, 0))

    @functools.partial(
        pl.kernel, out_type=out_type, mesh=mesh,
        compiler_params=pltpu.CompilerParams(use_tc_tiling_on_sc=False))
    def gk(mu_idx_h, rho_idx_h, nz_idx_h, t_h, mu_h, rho_h, nz_h):
        def body(i_v, o_v):
            pltpu.sync_copy(t_h.at[i_v.at[0]], o_v)

        pltpu.emit_pipeline(
            body, grid=(n // _WIN,),
            in_specs=[ispec], out_specs=[ospec],
            core_axis_name=("c", "s"), dimension_semantics=(pltpu.PARALLEL,),
        )(mu_idx_h, mu_h)
        pltpu.emit_pipeline(
            body, grid=(n // _WIN,),
            in_specs=[ispec], out_specs=[ospec],
            core_axis_name=("c", "s"), dimension_semantics=(pltpu.PARALLEL,),
        )(rho_idx_h, rho_h)
        pltpu.emit_pipeline(
            body, grid=(n_nz // _WIN,),
            in_specs=[ispec], out_specs=[ospec],
            core_axis_name=("c", "s"), dimension_semantics=(pltpu.PARALLEL,),
        )(nz_idx_h, nz_h)

    return gk(mu_idx, rho_idx, nz_idx, tab)


def _sc_gather_in(mu_idx, rho_idx, tab):
    n = mu_idx.shape[1]
    D = tab.shape[1]
    f32 = jnp.float32
    mesh = plsc.VectorSubcoreMesh(core_axis_name="c", subcore_axis_name="s")
    out_type = [jax.ShapeDtypeStruct((n, D), f32),
                jax.ShapeDtypeStruct((n, D), f32)]
    ispec = pl.BlockSpec((1, _WIN), lambda i: (0, i))
    ospec = pl.BlockSpec((_WIN, D), lambda i: (i, 0))

    @functools.partial(
        pl.kernel, out_type=out_type, mesh=mesh,
        compiler_params=pltpu.CompilerParams(use_tc_tiling_on_sc=False))
    def gk(mu_idx_h, rho_idx_h, t_h, mu_h, rho_h):
        def body(i_v, o_v):
            pltpu.sync_copy(t_h.at[i_v.at[0]], o_v)

        pltpu.emit_pipeline(
            body, grid=(n // _WIN,),
            in_specs=[ispec], out_specs=[ospec],
            core_axis_name=("c", "s"), dimension_semantics=(pltpu.PARALLEL,),
        )(mu_idx_h, mu_h)
        pltpu.emit_pipeline(
            body, grid=(n // _WIN,),
            in_specs=[ispec], out_specs=[ospec],
            core_axis_name=("c", "s"), dimension_semantics=(pltpu.PARALLEL,),
        )(rho_idx_h, rho_h)

    return gk(mu_idx, rho_idx, tab)


def _tc_math(mu_in, rho_in, eps_in, covf, covw, wT, bvec,
             mu_p, rho_p, eps_p, noise_p, B, W, D):
    GB = 128
    nblocks = B // GB
    GP = GB * W // 2
    f32 = jnp.float32
    hi = lax.Precision.HIGHEST
    halfw = W // 2

    def body(mu_in_r, rho_in_r, eps_in_r, cov_r, covw_r, wT_r, b_r,
             mu_p_r, rho_p_r, eps_p_r, noise_r, kl_r, lik_r):
        mu_in = mu_in_r[...]
        rho_in = rho_in_r[...]
        eps_in = eps_in_r[...]
        cov = cov_r[...]
        covw = covw_r[...]
        wT = wT_r[...]
        bb = b_r[...]

        y = covw[0:1, :] + cov * (covw[1:2, :] - covw[0:1, :])
        sig_in = jnp.log(jnp.exp(rho_in) + 1.0)
        h = (jnp.dot(mu_in, wT[0:D, :], precision=hi, preferred_element_type=f32)
             + jnp.dot(y, wT[D:2 * D, :], precision=hi, preferred_element_type=f32)
             + bb)
        w_in = jnp.tanh(h) + _SCALING * sig_in * eps_in
        post_in = -0.5 * jnp.sum(eps_in * eps_in) - jnp.sum(jnp.log(sig_in))
        wsq = w_in * w_in
        prior_in = jnp.sum(jnp.log(0.5 * jnp.exp(-wsq / 2.0)
                                   + 0.5 * jnp.exp(-wsq / 0.08)))
        kl = W * (post_in - prior_in)

        mo = mu_p_r[...]
        ro = rho_p_r[...]
        ep = eps_p_r[...]
        sig_o = jnp.log(jnp.exp(ro) + 1.0)
        w_o = mo + _SCALING * sig_o * ep
        post_out = -0.5 * jnp.sum(ep * ep) - jnp.sum(jnp.log(sig_o))
        wsq_o = w_o * w_o
        prior_out = jnp.sum(jnp.log(0.5 * jnp.exp(-wsq_o / 2.0)
                                    + 0.5 * jnp.exp(-wsq_o / 0.08)))
        kl += post_out - prior_out

        rowi = lax.broadcasted_iota(jnp.int32, (GP, GB), 0) // halfw
        colj = lax.broadcasted_iota(jnp.int32, (GP, GB), 1)
        sel = (rowi == colj).astype(f32)
        wsel = jnp.dot(sel, w_in, precision=hi, preferred_element_type=f32)
        wp = jnp.concatenate([wsel, wsel], axis=1)
        prodt = wp * w_o
        sL = jnp.sum(prodt[:, 0:D], axis=1, keepdims=True)
        sR = jnp.sum(prodt[:, D:2 * D], axis=1, keepdims=True)
        lik = (jnp.sum(jnp.log(jax.nn.sigmoid(sL)))
               + jnp.sum(jnp.log(jax.nn.sigmoid(sR))))

        nz = noise_r[...]
        half = _NEGS * D
        ls = jnp.float32(0.0)
        for j in range(_NEGS):
            pair = jnp.concatenate([nz[:, j * D:(j + 1) * D],
                                    nz[:, half + j * D:half + (j + 1) * D]],
                                   axis=1)
            prodn = wp * pair
            nL = jnp.sum(prodn[:, 0:D], axis=1, keepdims=True)
            nR = jnp.sum(prodn[:, D:2 * D], axis=1, keepdims=True)
            ls += (jnp.sum(jnp.log(jax.nn.sigmoid(-nL)))
                   + jnp.sum(jnp.log(jax.nn.sigmoid(-nR))))
        lik += ls / _NEGS

        kl_r[...] = kl.reshape(1, 1, 1)
        lik_r[...] = lik.reshape(1, 1, 1)

    part_spec = pl.BlockSpec((1, 1, 1), lambda i: (i, 0, 0))
    kl_parts, lik_parts = pl.pallas_call(
        body,
        grid=(nblocks,),
        in_specs=[
            pl.BlockSpec((GB, D), lambda i: (i, 0)),
            pl.BlockSpec((GB, D), lambda i: (i, 0)),
            pl.BlockSpec((GB, D), lambda i: (i, 0)),
            pl.BlockSpec((GB, 1), lambda i: (i, 0)),
            pl.BlockSpec((2, D), lambda i: (0, 0)),
            pl.BlockSpec((2 * D, D), lambda i: (0, 0)),
            pl.BlockSpec((1, D), lambda i: (0, 0)),
            pl.BlockSpec((GP, 2 * D), lambda i: (i, 0)),
            pl.BlockSpec((GP, 2 * D), lambda i: (i, 0)),
            pl.BlockSpec((GP, 2 * D), lambda i: (i, 0)),
            pl.BlockSpec((GP, 2 * _NEGS * D), lambda i: (i, 0)),
        ],
        out_specs=[part_spec, part_spec],
        out_shape=[jax.ShapeDtypeStruct((nblocks, 1, 1), f32)] * 2,
        compiler_params=_PAR,
    )(mu_in, rho_in, eps_in, covf, covw, wT, bvec,
      mu_p, rho_p, eps_p, noise_p)
    return kl_parts, lik_parts


def kernel(inputs, outputs, covars, wt, batch_num, in_embed_w, out_embed_w,
           in_rho_w, out_rho_w, covariates_w, linear_w, linear_b):
    B, W = outputs.shape
    V, D = in_embed_w.shape

    key = jax.random.key(42)
    k1, k2, k3 = jax.random.split(key, 3)
    eps_in = jax.random.normal(k1, (B, D), jnp.float32)
    eps_p = jax.random.normal(k2, (B * W // 2, 2 * D), jnp.float32)
    noise_idx = jax.random.randint(k3, (B * W, _NEGS), 0, V)

    tab_out = _tc_transpose_fused(out_embed_w.T, out_rho_w.T)
    tab_in = _tc_transpose_fused(in_embed_w.T, in_rho_w.T)
    V2 = 2 * tab_out.shape[0]
    tab_out64 = tab_out.reshape(V2, D)
    tab_in64 = tab_in.reshape(V2, D)

    o2 = 2 * outputs.astype(jnp.int32).reshape(1, B * W)
    nz2 = 2 * noise_idx.astype(jnp.int32).reshape(1, B * W * _NEGS)
    i2 = 2 * inputs.astype(jnp.int32).reshape(1, B)

    mu_out_d, rho_out_d, noise_d = _sc_gather_out(o2, o2 + 1, nz2, tab_out64)
    mu_in_d, rho_in_d = _sc_gather_in(i2, i2 + 1, tab_in64)

    mu_p = mu_out_d.reshape(B * W // 2, 2 * D)
    rho_p = rho_out_d.reshape(B * W // 2, 2 * D)
    noise_p = noise_d.reshape(B * W // 2, 2 * _NEGS * D)

    covf = covars.astype(jnp.float32).reshape(B, 1)
    wT = linear_w.T
    bvec = linear_b.reshape(1, D)

    kl_parts, lik_parts = _tc_math(mu_in_d, rho_in_d, eps_in, covf,
                                   covariates_w, wT, bvec, mu_p, rho_p,
                                   eps_p, noise_p, B, W, D)
    loss = (wt[0] * jnp.sum(kl_parts) - jnp.sum(lik_parts)) / (B * W)
    return loss


# speedup vs baseline: 2.4467x; 1.2499x over previous
import functools

import jax
import jax.numpy as jnp
from jax import lax
from jax.experimental import pallas as pl
from jax.experimental.pallas import tpu as pltpu
from jax.experimental.pallas import tpu_sc as plsc

_NEGS = 5
_SCALING = 0.1
_WIN = 128
_TBK = 8192

_PAR = pltpu.CompilerParams(dimension_semantics=("parallel",))


def _tc_transpose_fused(at, bt):
    D, V = at.shape
    grid = (V + _TBK - 1) // _TBK

    def body(a_r, b_r, o_r):
        x = jnp.concatenate([a_r[...], b_r[...]], axis=0)
        o_r[...] = jnp.transpose(x, (1, 0))

    return pl.pallas_call(
        body,
        grid=(grid,),
        in_specs=[pl.BlockSpec((D, _TBK), lambda i: (0, i)),
                  pl.BlockSpec((D, _TBK), lambda i: (0, i))],
        out_specs=pl.BlockSpec((_TBK, 2 * D), lambda i: (i, 0)),
        out_shape=jax.ShapeDtypeStruct((grid * _TBK, 2 * D), jnp.float32),
        compiler_params=_PAR,
    )(at, bt)


def _sc_gather_out(mu_idx, rho_idx, nz_idx, tab):
    n = mu_idx.shape[1]
    n_nz = nz_idx.shape[1]
    D = tab.shape[1]
    f32 = jnp.float32
    mesh = plsc.VectorSubcoreMesh(core_axis_name="c", subcore_axis_name="s")
    out_type = [jax.ShapeDtypeStruct((n, D), f32),
                jax.ShapeDtypeStruct((n, D), f32),
                jax.ShapeDtypeStruct((n_nz, D), f32)]
    ispec = pl.BlockSpec((1, _WIN), lambda i: (0, i))
    ospec = pl.BlockSpec((_WIN, D), lambda i: (i, 0))

    @functools.partial(
        pl.kernel, out_type=out_type, mesh=mesh,
        compiler_params=pltpu.CompilerParams(use_tc_tiling_on_sc=False))
    def gk(mu_idx_h, rho_idx_h, nz_idx_h, t_h, mu_h, rho_h, nz_h):
        def body(i_v, o_v):
            pltpu.sync_copy(t_h.at[i_v.at[0]], o_v)

        pltpu.emit_pipeline(
            body, grid=(n // _WIN,),
            in_specs=[ispec], out_specs=[ospec],
            core_axis_name=("c", "s"), dimension_semantics=(pltpu.PARALLEL,),
        )(mu_idx_h, mu_h)
        pltpu.emit_pipeline(
            body, grid=(n // _WIN,),
            in_specs=[ispec], out_specs=[ospec],
            core_axis_name=("c", "s"), dimension_semantics=(pltpu.PARALLEL,),
        )(rho_idx_h, rho_h)
        pltpu.emit_pipeline(
            body, grid=(n_nz // _WIN,),
            in_specs=[ispec], out_specs=[ospec],
            core_axis_name=("c", "s"), dimension_semantics=(pltpu.PARALLEL,),
        )(nz_idx_h, nz_h)

    return gk(mu_idx, rho_idx, nz_idx, tab)


def _sc_gather_in(mu_idx, rho_idx, tab):
    n = mu_idx.shape[1]
    D = tab.shape[1]
    f32 = jnp.float32
    mesh = plsc.VectorSubcoreMesh(core_axis_name="c", subcore_axis_name="s")
    out_type = [jax.ShapeDtypeStruct((n, D), f32),
                jax.ShapeDtypeStruct((n, D), f32)]
    ispec = pl.BlockSpec((1, _WIN), lambda i: (0, i))
    ospec = pl.BlockSpec((_WIN, D), lambda i: (i, 0))

    @functools.partial(
        pl.kernel, out_type=out_type, mesh=mesh,
        compiler_params=pltpu.CompilerParams(use_tc_tiling_on_sc=False))
    def gk(mu_idx_h, rho_idx_h, t_h, mu_h, rho_h):
        def body(i_v, o_v):
            pltpu.sync_copy(t_h.at[i_v.at[0]], o_v)

        pltpu.emit_pipeline(
            body, grid=(n // _WIN,),
            in_specs=[ispec], out_specs=[ospec],
            core_axis_name=("c", "s"), dimension_semantics=(pltpu.PARALLEL,),
        )(mu_idx_h, mu_h)
        pltpu.emit_pipeline(
            body, grid=(n // _WIN,),
            in_specs=[ispec], out_specs=[ospec],
            core_axis_name=("c", "s"), dimension_semantics=(pltpu.PARALLEL,),
        )(rho_idx_h, rho_h)

    return gk(mu_idx, rho_idx, tab)


def _tc_math(mu_in, rho_in, eps_in, covf, covw, wT, bvec,
             mu_p, rho_p, eps_p, noise_p, B, W, D):
    GB = 128
    nblocks = B // GB
    GP = GB * W // 2
    f32 = jnp.float32
    hi = lax.Precision.HIGHEST
    halfw = W // 2

    def body(mu_in_r, rho_in_r, eps_in_r, cov_r, covw_r, wT_r, b_r,
             mu_p_r, rho_p_r, eps_p_r, noise_r, kl_r, lik_r):
        mu_in = mu_in_r[...]
        rho_in = rho_in_r[...]
        eps_in = eps_in_r[...]
        cov = cov_r[...]
        covw = covw_r[...]
        wT = wT_r[...]
        bb = b_r[...]

        y = covw[0:1, :] + cov * (covw[1:2, :] - covw[0:1, :])
        sig_in = jnp.log(jnp.exp(rho_in) + 1.0)
        h = (jnp.dot(mu_in, wT[0:D, :], precision=hi, preferred_element_type=f32)
             + jnp.dot(y, wT[D:2 * D, :], precision=hi, preferred_element_type=f32)
             + bb)
        w_in = jnp.tanh(h) + _SCALING * sig_in * eps_in
        post_in = -0.5 * jnp.sum(eps_in * eps_in) - jnp.sum(jnp.log(sig_in))
        wsq = w_in * w_in
        prior_in = jnp.sum(jnp.log(0.5 * jnp.exp(-wsq / 2.0)
                                   + 0.5 * jnp.exp(-wsq / 0.08)))
        kl = W * (post_in - prior_in)

        mo = mu_p_r[...]
        ro = rho_p_r[...]
        ep = eps_p_r[...]
        sig_o = jnp.log(jnp.exp(ro) + 1.0)
        w_o = mo + _SCALING * sig_o * ep
        post_out = -0.5 * jnp.sum(ep * ep) - jnp.sum(jnp.log(sig_o))
        wsq_o = w_o * w_o
        prior_out = jnp.sum(jnp.log(0.5 * jnp.exp(-wsq_o / 2.0)
                                    + 0.5 * jnp.exp(-wsq_o / 0.08)))
        kl += post_out - prior_out

        rowi = lax.broadcasted_iota(jnp.int32, (GP, GB), 0) // halfw
        colj = lax.broadcasted_iota(jnp.int32, (GP, GB), 1)
        sel = (rowi == colj).astype(f32)
        wsel = jnp.dot(sel, w_in, precision=hi, preferred_element_type=f32)
        wp = jnp.concatenate([wsel, wsel], axis=1)
        prodt = wp * w_o
        sL = jnp.sum(prodt[:, 0:D], axis=1, keepdims=True)
        sR = jnp.sum(prodt[:, D:2 * D], axis=1, keepdims=True)
        lik = (jnp.sum(jnp.log(jax.nn.sigmoid(sL)))
               + jnp.sum(jnp.log(jax.nn.sigmoid(sR))))

        nz = noise_r[...]
        half = _NEGS * D
        ls = jnp.float32(0.0)
        for j in range(_NEGS):
            pair = jnp.concatenate([nz[:, j * D:(j + 1) * D],
                                    nz[:, half + j * D:half + (j + 1) * D]],
                                   axis=1)
            prodn = wp * pair
            nL = jnp.sum(prodn[:, 0:D], axis=1, keepdims=True)
            nR = jnp.sum(prodn[:, D:2 * D], axis=1, keepdims=True)
            ls += (jnp.sum(jnp.log(jax.nn.sigmoid(-nL)))
                   + jnp.sum(jnp.log(jax.nn.sigmoid(-nR))))
        lik += ls / _NEGS

        kl_r[...] = kl.reshape(1, 1, 1)
        lik_r[...] = lik.reshape(1, 1, 1)

    part_spec = pl.BlockSpec((1, 1, 1), lambda i: (i, 0, 0))
    kl_parts, lik_parts = pl.pallas_call(
        body,
        grid=(nblocks,),
        in_specs=[
            pl.BlockSpec((GB, D), lambda i: (i, 0)),
            pl.BlockSpec((GB, D), lambda i: (i, 0)),
            pl.BlockSpec((GB, D), lambda i: (i, 0)),
            pl.BlockSpec((GB, 1), lambda i: (i, 0)),
            pl.BlockSpec((2, D), lambda i: (0, 0)),
            pl.BlockSpec((2 * D, D), lambda i: (0, 0)),
            pl.BlockSpec((1, D), lambda i: (0, 0)),
            pl.BlockSpec((GP, 2 * D), lambda i: (i, 0)),
            pl.BlockSpec((GP, 2 * D), lambda i: (i, 0)),
            pl.BlockSpec((GP, 2 * D), lambda i: (i, 0)),
            pl.BlockSpec((GP, 2 * _NEGS * D), lambda i: (i, 0)),
        ],
        out_specs=[part_spec, part_spec],
        out_shape=[jax.ShapeDtypeStruct((nblocks, 1, 1), f32)] * 2,
        compiler_params=_PAR,
    )(mu_in, rho_in, eps_in, covf, covw, wT, bvec,
      mu_p, rho_p, eps_p, noise_p)
    return kl_parts, lik_parts


def kernel(inputs, outputs, covars, wt, batch_num, in_embed_w, out_embed_w,
           in_rho_w, out_rho_w, covariates_w, linear_w, linear_b):
    B, W = outputs.shape
    V, D = in_embed_w.shape

    key = jax.random.key(42)
    k1, k2, k3 = jax.random.split(key, 3)
    eps_in = jax.random.normal(k1, (B, D), jnp.float32)
    eps_p = jax.random.normal(k2, (B * W // 2, 2 * D), jnp.float32)
    noise_idx = jax.random.randint(k3, (B * W, _NEGS), 0, V)

    tab_out = _tc_transpose_fused(out_embed_w.T, out_rho_w.T)
    tab_in = _tc_transpose_fused(in_embed_w.T, in_rho_w.T)
    V2 = 2 * tab_out.shape[0]
    tab_out64 = tab_out.reshape(V2, D)
    tab_in64 = tab_in.reshape(V2, D)

    o2 = 2 * outputs.astype(jnp.int32).reshape(1, B * W)
    nz2 = 2 * noise_idx.astype(jnp.int32).reshape(1, B * W * _NEGS)
    i2 = 2 * inputs.astype(jnp.int32).reshape(1, B)

    mu_out_d, rho_out_d, noise_d = _sc_gather_out(o2, o2 + 1, nz2, tab_out64)
    mu_in_d, rho_in_d = _sc_gather_in(i2, i2 + 1, tab_in64)

    mu_p = mu_out_d.reshape(B * W // 2, 2 * D)
    rho_p = rho_out_d.reshape(B * W // 2, 2 * D)
    noise_p = noise_d.reshape(B * W // 2, 2 * _NEGS * D)

    covf = covars.astype(jnp.float32).reshape(B, 1)
    wT = linear_w.T
    bvec = linear_b.reshape(1, D)

    kl_parts, lik_parts = _tc_math(mu_in_d, rho_in_d, eps_in, covf,
                                   covariates_w, wT, bvec, mu_p, rho_p,
                                   eps_p, noise_p, B, W, D)
    loss = (wt[0] * jnp.sum(kl_parts) - jnp.sum(lik_parts)) / (B * W)
    return loss
